# Initial kernel scaffold; baseline (speedup 1.0000x reference)
import jax
import jax.numpy as jnp
from jax.experimental import pallas as pl


def kernel(adj_indices, adj_values, embeds):
    raise NotImplementedError("write your pallas kernel here")


# speedup vs baseline: 4.0919x; 4.0919x over previous
import functools

import jax
import jax.numpy as jnp
from jax import lax
from jax.experimental import pallas as pl
from jax.experimental.pallas import tpu as pltpu
from jax.experimental.pallas import tpu_sc as plsc

N = 10000
E = 320000
D = 128
LEAKY = 0.2

NC = 2
NS = 16
NW = NC * NS
EPW = E // NW
CHUNK = 80
NCHUNKS = EPW // CHUNK
NPAD = 10240
RPT = NPAD // NS


@functools.cache
def _build_spmm():
    mesh = plsc.VectorSubcoreMesh(core_axis_name="c", subcore_axis_name="s")

    @functools.partial(
        pl.kernel,
        out_type=jax.ShapeDtypeStruct((NC, N, D), jnp.float32),
        mesh=mesh,
        scratch_types=[
            pltpu.VMEM((CHUNK,), jnp.int32),
            pltpu.VMEM((CHUNK,), jnp.int32),
            pltpu.VMEM((CHUNK,), jnp.float32),
            pltpu.VMEM((CHUNK, D), jnp.float32),
            pltpu.VMEM_SHARED((NPAD, D), jnp.float32),
            pltpu.SemaphoreType.DMA,
        ],
        compiler_params=pltpu.CompilerParams(needs_layout_passes=False),
    )
    def spmm(src_hbm, dst_hbm, val_hbm, emb_hbm, out_hbm,
             src_v, dst_v, val_v, rows_v, acc, sem):
        c = lax.axis_index("c")
        s = lax.axis_index("s")

        def zero_row(r, carry):
            for k in range(D // 16):
                rows_v[r, pl.ds(k * 16, 16)] = jnp.zeros((16,), jnp.float32)
            return carry

        lax.fori_loop(0, CHUNK, zero_row, 0)
        for j in range(RPT // CHUNK):
            pltpu.sync_copy(rows_v, acc.at[pl.ds(s * RPT + j * CHUNK, CHUNK)])
        plsc.subcore_barrier()

        base = (s * NC + c) * EPW

        def do_chunk(j, carry):
            eb = base + j * CHUNK
            pltpu.sync_copy(src_hbm.at[pl.ds(eb, CHUNK)], src_v)
            pltpu.sync_copy(dst_hbm.at[pl.ds(eb, CHUNK)], dst_v)
            pltpu.sync_copy(val_hbm.at[pl.ds(eb, CHUNK)], val_v)
            pltpu.async_copy(emb_hbm.at[src_v], rows_v, sem).wait()

            def scale_row(r, rcarry):
                vsplat = plsc.load_gather(
                    val_v, [jnp.full((16,), r, jnp.int32)])
                for k in range(D // 16):
                    sl = pl.ds(k * 16, 16)
                    rows_v[r, sl] = rows_v[r, sl] * vsplat
                return rcarry

            lax.fori_loop(0, CHUNK, scale_row, 0)

            pltpu.sync_copy(rows_v, acc.at[dst_v], add=True)
            return carry

        lax.fori_loop(0, NCHUNKS, do_chunk, 0)
        plsc.subcore_barrier()

        for j in range(RPT // CHUNK):
            start = s * RPT + j * CHUNK

            @pl.when(start < N)
            def _():
                pltpu.sync_copy(acc.at[pl.ds(start, CHUNK)],
                                out_hbm.at[c, pl.ds(start, CHUNK)])

    return spmm


_COMBINE_BLK = 2000


def _combine_body(p_ref, o_ref):
    x = p_ref[0] + p_ref[1]
    o_ref[...] = jnp.where(x >= 0, x, LEAKY * x)


@functools.cache
def _build_combine():
    return pl.pallas_call(
        _combine_body,
        grid=(N // _COMBINE_BLK,),
        in_specs=[pl.BlockSpec((NC, _COMBINE_BLK, D), lambda i: (0, i, 0))],
        out_specs=pl.BlockSpec((_COMBINE_BLK, D), lambda i: (i, 0)),
        out_shape=jax.ShapeDtypeStruct((N, D), jnp.float32),
    )


def kernel(adj_indices, adj_values, embeds):
    adj_indices = adj_indices.astype(jnp.int32)
    dst = adj_indices[0]
    src = adj_indices[1]
    partials = _build_spmm()(src, dst, adj_values, embeds)
    return _build_combine()(partials)


# speedup vs baseline: 6.6890x; 1.6347x over previous
import functools

import jax
import jax.numpy as jnp
from jax import lax
from jax.experimental import pallas as pl
from jax.experimental.pallas import tpu as pltpu
from jax.experimental.pallas import tpu_sc as plsc

N = 10000
E = 320000
D = 128
LEAKY = 0.2

NC = 2
NS = 16
NW = NC * NS
EPW = E // NW
CHUNK = 80
NCHUNKS = EPW // CHUNK
NPAD = 10240
RPT = NPAD // NS


@functools.cache
def _build_spmm():
    mesh = plsc.VectorSubcoreMesh(core_axis_name="c", subcore_axis_name="s")

    @functools.partial(
        pl.kernel,
        out_type=jax.ShapeDtypeStruct((NC, NPAD, D), jnp.float32),
        mesh=mesh,
        scratch_types=[
            pltpu.VMEM((3, CHUNK), jnp.int32),
            pltpu.VMEM((3, CHUNK), jnp.int32),
            pltpu.VMEM((CHUNK, D), jnp.float32),
            pltpu.VMEM((CHUNK, D), jnp.float32),
            pltpu.VMEM_SHARED((NPAD, D), jnp.float32),
            pltpu.SemaphoreType.DMA,
            pltpu.SemaphoreType.DMA,
        ],
        compiler_params=pltpu.CompilerParams(needs_layout_passes=False),
    )
    def spmm(edges_hbm, emb_hbm, out_hbm,
             ebuf0, ebuf1, rows0, rows1, acc, sem_e, sem_g):
        c = lax.axis_index("c")
        s = lax.axis_index("s")
        wid = s * NC + c

        def zero_row(r, carry):
            for k in range(D // 16):
                rows0[r, pl.ds(k * 16, 16)] = jnp.zeros((16,), jnp.float32)
            return carry

        lax.fori_loop(0, CHUNK, zero_row, 0)
        for j in range(RPT // CHUNK):
            pltpu.sync_copy(rows0, acc.at[pl.ds(s * RPT + j * CHUNK, CHUNK)])
        plsc.subcore_barrier()

        ebufs = [ebuf0, ebuf1]
        rows = [rows0, rows1]

        def process(j, ebuf, buf):
            pltpu.make_async_copy(emb_hbm.at[ebuf.at[0]], buf, sem_g).wait()

            def scale_row(r, rcarry):
                vbits = plsc.load_gather(
                    ebuf,
                    [jnp.full((16,), 2, jnp.int32),
                     jnp.full((16,), r, jnp.int32)])
                vsplat = plsc.bitcast(vbits, jnp.float32)
                for k in range(D // 16):
                    sl = pl.ds(k * 16, 16)
                    buf[r, sl] = buf[r, sl] * vsplat
                return rcarry

            lax.fori_loop(0, CHUNK, scale_row, 0)

            pltpu.sync_copy(buf, acc.at[ebuf.at[1]], add=True)

        pltpu.async_copy(edges_hbm.at[wid, 0], ebuf0, sem_e)
        pltpu.make_async_copy(edges_hbm.at[wid, 0], ebuf0, sem_e).wait()
        pltpu.async_copy(emb_hbm.at[ebuf0.at[0]], rows0, sem_g)
        pltpu.async_copy(edges_hbm.at[wid, 1], ebuf1, sem_e)

        def pair(p, carry):
            for b in range(2):
                j = 2 * p + b
                eb_q, rows_q = ebufs[1 - b], rows[1 - b]
                eb_p, rows_p = ebufs[b], rows[b]
                pltpu.make_async_copy(
                    edges_hbm.at[wid, j + 1], eb_q, sem_e).wait()
                pltpu.async_copy(emb_hbm.at[eb_q.at[0]], rows_q, sem_g)
                process(j, eb_p, rows_p)

                @pl.when(j < NCHUNKS - 2)
                def _():
                    pltpu.async_copy(edges_hbm.at[wid, j + 2], eb_p, sem_e)
            return carry

        lax.fori_loop(0, (NCHUNKS - 1) // 2, pair, 0)
        tail = NCHUNKS - 1
        process(tail, ebufs[tail % 2], rows[tail % 2])
        plsc.subcore_barrier()

        for j in range(RPT // CHUNK):
            start = s * RPT + j * CHUNK
            pltpu.sync_copy(acc.at[pl.ds(start, CHUNK)],
                            out_hbm.at[c, pl.ds(start, CHUNK)])

    return spmm


_COMBINE_BLK = 2048


def _combine_body(p_ref, o_ref):
    x = p_ref[0] + p_ref[1]
    o_ref[...] = jnp.where(x >= 0, x, LEAKY * x)


@functools.cache
def _build_combine():
    return pl.pallas_call(
        _combine_body,
        grid=(NPAD // _COMBINE_BLK,),
        in_specs=[pl.BlockSpec((NC, _COMBINE_BLK, D), lambda i: (0, i, 0))],
        out_specs=pl.BlockSpec((_COMBINE_BLK, D), lambda i: (i, 0)),
        out_shape=jax.ShapeDtypeStruct((NPAD, D), jnp.float32),
    )


def kernel(adj_indices, adj_values, embeds):
    adj_indices = adj_indices.astype(jnp.int32)
    dst = adj_indices[0].reshape(NW, NCHUNKS, CHUNK)
    src = adj_indices[1].reshape(NW, NCHUNKS, CHUNK)
    vals = lax.bitcast_convert_type(
        adj_values, jnp.int32).reshape(NW, NCHUNKS, CHUNK)
    edges = jnp.stack([src, dst, vals], axis=2)
    partials = _build_spmm()(edges, embeds)
    return _build_combine()(partials)[:N]


# speedup vs baseline: 9.5165x; 1.4227x over previous
import functools

import jax
import jax.numpy as jnp
from jax import lax
from jax.experimental import pallas as pl
from jax.experimental.pallas import tpu as pltpu
from jax.experimental.pallas import tpu_sc as plsc

N = 10000
E = 320000
D = 128
LEAKY = 0.2

NC = 2
NS = 16
NW = NC * NS
EPW = E // NW
CHUNK = 80
NCHUNKS = EPW // CHUNK
NPAD = 10240
RPT = NPAD // NS
UNROLL = 6
BODY_ITERS = (NCHUNKS - 1) // UNROLL
TAIL_START = 1 + BODY_ITERS * UNROLL


@functools.cache
def _build_spmm():
    mesh = plsc.VectorSubcoreMesh(core_axis_name="c", subcore_axis_name="s")

    @functools.partial(
        pl.kernel,
        out_type=jax.ShapeDtypeStruct((NC, NPAD, D), jnp.float32),
        mesh=mesh,
        scratch_types=[
            [pltpu.VMEM((CHUNK,), jnp.int32) for _ in range(3)],
            [pltpu.VMEM((CHUNK,), jnp.int32) for _ in range(3)],
            [pltpu.VMEM((CHUNK,), jnp.float32) for _ in range(3)],
            [pltpu.VMEM((CHUNK, D), jnp.float32) for _ in range(2)],
            pltpu.VMEM_SHARED((NPAD, D), jnp.float32),
            pltpu.SemaphoreType.DMA,
            pltpu.SemaphoreType.DMA,
            pltpu.SemaphoreType.DMA,
        ],
        compiler_params=pltpu.CompilerParams(needs_layout_passes=False),
    )
    def spmm(src_hbm, dst_hbm, val_hbm, emb_hbm, out_hbm,
             srcs, dsts, vals, rows, acc, sem_e, sem_g, sem_s):
        c = lax.axis_index("c")
        s = lax.axis_index("s")
        base = (s * NC + c) * EPW

        def zero_row(r, carry):
            for k in range(D // 16):
                rows[0][r, pl.ds(k * 16, 16)] = jnp.zeros((16,), jnp.float32)
            return carry

        lax.fori_loop(0, CHUNK, zero_row, 0)
        for j in range(RPT // CHUNK):
            pltpu.sync_copy(rows[0],
                            acc.at[pl.ds(s * RPT + j * CHUNK, CHUNK)])
        plsc.subcore_barrier()

        def issue_idx(j, t):
            eb = base + j * CHUNK
            pltpu.async_copy(src_hbm.at[pl.ds(eb, CHUNK)], srcs[t], sem_e)
            pltpu.async_copy(dst_hbm.at[pl.ds(eb, CHUNK)], dsts[t], sem_e)
            pltpu.async_copy(val_hbm.at[pl.ds(eb, CHUNK)], vals[t], sem_e)

        def wait_idx(t):
            pltpu.make_async_copy(
                src_hbm.at[pl.ds(0, CHUNK)], srcs[t], sem_e).wait()
            pltpu.make_async_copy(
                dst_hbm.at[pl.ds(0, CHUNK)], dsts[t], sem_e).wait()
            pltpu.make_async_copy(
                val_hbm.at[pl.ds(0, CHUNK)], vals[t], sem_e).wait()

        def wait_scatter(b, t):
            pltpu.make_async_copy(rows[b], acc.at[dsts[t]], sem_s).wait()

        def step(j, b, t, *, first=False, issue_next=True, prefetch=True):
            if issue_next:
                wait_idx((t + 1) % 3)
            if not first:
                wait_scatter(1 - b, (t + 2) % 3)
            if issue_next:
                pltpu.async_copy(
                    emb_hbm.at[srcs[(t + 1) % 3]], rows[1 - b], sem_g)
            if prefetch:
                issue_idx(j + 2, (t + 2) % 3)
            pltpu.make_async_copy(emb_hbm.at[srcs[t]], rows[b], sem_g).wait()

            def scale_row(r, rcarry):
                vsplat = plsc.load_gather(
                    vals[t], [jnp.full((16,), r, jnp.int32)])
                for k in range(D // 16):
                    sl = pl.ds(k * 16, 16)
                    rows[b][r, sl] = rows[b][r, sl] * vsplat
                return rcarry

            lax.fori_loop(0, CHUNK, scale_row, 0)

            pltpu.async_copy(rows[b], acc.at[dsts[t]], sem_s, add=True)

        issue_idx(0, 0)
        wait_idx(0)
        pltpu.async_copy(emb_hbm.at[srcs[0]], rows[0], sem_g)
        issue_idx(1, 1)

        step(0, 0, 0, first=True)

        def six(p, carry):
            for u in range(UNROLL):
                j = (1 + u) + p * UNROLL
                step(j, (1 + u) % 2, (1 + u) % 3)
            return carry

        lax.fori_loop(0, BODY_ITERS, six, 0)

        for j in range(TAIL_START, NCHUNKS):
            step(j, j % 2, j % 3,
                 issue_next=(j + 1 < NCHUNKS),
                 prefetch=(j + 2 < NCHUNKS))
        wait_scatter((NCHUNKS - 1) % 2, (NCHUNKS - 1) % 3)
        plsc.subcore_barrier()

        for j in range(RPT // CHUNK):
            start = s * RPT + j * CHUNK
            pltpu.sync_copy(acc.at[pl.ds(start, CHUNK)],
                            out_hbm.at[c, pl.ds(start, CHUNK)])

    return spmm


_COMBINE_BLK = 2048


def _combine_body(p_ref, o_ref):
    x = p_ref[0] + p_ref[1]
    o_ref[...] = jnp.where(x >= 0, x, LEAKY * x)


@functools.cache
def _build_combine():
    return pl.pallas_call(
        _combine_body,
        grid=(NPAD // _COMBINE_BLK,),
        in_specs=[pl.BlockSpec((NC, _COMBINE_BLK, D), lambda i: (0, i, 0))],
        out_specs=pl.BlockSpec((_COMBINE_BLK, D), lambda i: (i, 0)),
        out_shape=jax.ShapeDtypeStruct((NPAD, D), jnp.float32),
    )


def kernel(adj_indices, adj_values, embeds):
    adj_indices = adj_indices.astype(jnp.int32)
    dst = adj_indices[0]
    src = adj_indices[1]
    partials = _build_spmm()(src, dst, adj_values, embeds)
    return _build_combine()(partials)[:N]


# speedup vs baseline: 11.8687x; 1.2472x over previous
import functools

import jax
import jax.numpy as jnp
from jax import lax
from jax.experimental import pallas as pl
from jax.experimental.pallas import tpu as pltpu
from jax.experimental.pallas import tpu_sc as plsc

N = 10000
E = 320000
D = 128
LEAKY = 0.2

NC = 2
NS = 16
NW = NC * NS
EPW = E // NW
CHUNK = 80
HALF = CHUNK // 2
NCHUNKS = EPW // CHUNK
NPAD = 10240
RPT = NPAD // NS
UNROLL = 6
BODY_ITERS = (NCHUNKS - 1) // UNROLL
TAIL_START = 1 + BODY_ITERS * UNROLL


@functools.cache
def _build_spmm():
    mesh = plsc.VectorSubcoreMesh(core_axis_name="c", subcore_axis_name="s")

    @functools.partial(
        pl.kernel,
        out_type=jax.ShapeDtypeStruct((NC, NPAD, D), jnp.float32),
        mesh=mesh,
        scratch_types=[
            [pltpu.VMEM((CHUNK,), jnp.int32) for _ in range(3)],
            [pltpu.VMEM((2, HALF), jnp.int32) for _ in range(3)],
            [pltpu.VMEM((CHUNK,), jnp.float32) for _ in range(3)],
            [pltpu.VMEM((CHUNK, D), jnp.float32) for _ in range(2)],
            pltpu.VMEM_SHARED((NPAD, D), jnp.float32),
            pltpu.SemaphoreType.DMA,
            pltpu.SemaphoreType.DMA,
            pltpu.SemaphoreType.DMA,
        ],
        compiler_params=pltpu.CompilerParams(needs_layout_passes=False),
    )
    def spmm(src_hbm, dst_hbm, val_hbm, emb_hbm, out_hbm,
             srcs, dsts, vals, rows, acc, sem_e, sem_g, sem_s):
        c = lax.axis_index("c")
        s = lax.axis_index("s")
        base = (s * NC + c) * EPW

        def zero_row(r, carry):
            for k in range(D // 16):
                rows[0][r, pl.ds(k * 16, 16)] = jnp.zeros((16,), jnp.float32)
            return carry

        lax.fori_loop(0, CHUNK, zero_row, 0)
        for j in range(RPT // CHUNK):
            pltpu.sync_copy(rows[0],
                            acc.at[pl.ds(s * RPT + j * CHUNK, CHUNK)])
        plsc.subcore_barrier()

        def issue_idx(j, t):
            eb = base + j * CHUNK
            pltpu.async_copy(src_hbm.at[pl.ds(eb, CHUNK)], srcs[t], sem_e)
            for h in range(2):
                pltpu.async_copy(dst_hbm.at[pl.ds(eb + h * HALF, HALF)],
                                 dsts[t].at[h], sem_e)
            pltpu.async_copy(val_hbm.at[pl.ds(eb, CHUNK)], vals[t], sem_e)

        def wait_idx(t):
            pltpu.make_async_copy(
                src_hbm.at[pl.ds(0, CHUNK)], srcs[t], sem_e).wait()
            for h in range(2):
                pltpu.make_async_copy(
                    dst_hbm.at[pl.ds(0, HALF)], dsts[t].at[h], sem_e).wait()
            pltpu.make_async_copy(
                val_hbm.at[pl.ds(0, CHUNK)], vals[t], sem_e).wait()

        def wait_scatter(b, t):
            for h in range(2):
                pltpu.make_async_copy(rows[b].at[pl.ds(h * HALF, HALF)],
                                      acc.at[dsts[t].at[h]], sem_s).wait()

        def step(j, b, t, *, first=False, issue_next=True, prefetch=True):
            if issue_next:
                wait_idx((t + 1) % 3)
            if not first:
                wait_scatter(1 - b, (t + 2) % 3)
            if issue_next:
                pltpu.async_copy(
                    emb_hbm.at[srcs[(t + 1) % 3]], rows[1 - b], sem_g)
            if prefetch:
                issue_idx(j + 2, (t + 2) % 3)
            pltpu.make_async_copy(emb_hbm.at[srcs[t]], rows[b], sem_g).wait()

            for h in range(2):
                @plsc.parallel_loop(h * HALF, (h + 1) * HALF, 1, unroll=2)
                def scale_row(r):
                    vsplat = plsc.load_gather(
                        vals[t], [jnp.full((16,), r, jnp.int32)])
                    for k in range(D // 16):
                        sl = pl.ds(k * 16, 16)
                        rows[b][r, sl] = rows[b][r, sl] * vsplat

                pltpu.async_copy(rows[b].at[pl.ds(h * HALF, HALF)],
                                 acc.at[dsts[t].at[h]], sem_s, add=True)

        issue_idx(0, 0)
        wait_idx(0)
        pltpu.async_copy(emb_hbm.at[srcs[0]], rows[0], sem_g)
        issue_idx(1, 1)

        step(0, 0, 0, first=True)

        def six(p, carry):
            for u in range(UNROLL):
                j = (1 + u) + p * UNROLL
                step(j, (1 + u) % 2, (1 + u) % 3)
            return carry

        lax.fori_loop(0, BODY_ITERS, six, 0)

        for j in range(TAIL_START, NCHUNKS):
            step(j, j % 2, j % 3,
                 issue_next=(j + 1 < NCHUNKS),
                 prefetch=(j + 2 < NCHUNKS))
        wait_scatter((NCHUNKS - 1) % 2, (NCHUNKS - 1) % 3)
        plsc.subcore_barrier()

        for j in range(RPT // CHUNK):
            start = s * RPT + j * CHUNK
            pltpu.sync_copy(acc.at[pl.ds(start, CHUNK)],
                            out_hbm.at[c, pl.ds(start, CHUNK)])

    return spmm


_COMBINE_BLK = 2000


def _combine_body(p_ref, o_ref):
    x = p_ref[0] + p_ref[1]
    o_ref[...] = jnp.where(x >= 0, x, LEAKY * x)


@functools.cache
def _build_combine():
    return pl.pallas_call(
        _combine_body,
        grid=(N // _COMBINE_BLK,),
        in_specs=[pl.BlockSpec((NC, _COMBINE_BLK, D), lambda i: (0, i, 0))],
        out_specs=pl.BlockSpec((_COMBINE_BLK, D), lambda i: (i, 0)),
        out_shape=jax.ShapeDtypeStruct((N, D), jnp.float32),
    )


def kernel(adj_indices, adj_values, embeds):
    adj_indices = adj_indices.astype(jnp.int32)
    dst = adj_indices[0]
    src = adj_indices[1]
    partials = _build_spmm()(src, dst, adj_values, embeds)
    return _build_combine()(partials)
